# baseline (speedup 1.0000x reference)
import functools

import jax
import jax.numpy as jnp
from jax import lax
from jax.experimental import pallas as pl
from jax.experimental.pallas import tpu as pltpu
from jax.experimental.pallas import tpu_sc as plsc

_NC = 2
_NS = 16
_NW = _NC * _NS
_CHUNK = 128
_NBUF = 4

_BQ = 8192
_NBQ = 16
_VQP2 = _BQ * _NBQ


def _rne_bf16_bits(y):
    u = lax.bitcast_convert_type(y, jnp.uint32)
    return (u + 0x7FFF + ((u >> 16) & 1)) >> 16


def _pack_kernel(x0, x1, x2, x3, x4, x5, x6, x7, out_ref):
    D = x0.shape[0]
    lane = lax.broadcasted_iota(jnp.int32, (D, 4 * D), 1)
    sub = lax.broadcasted_iota(jnp.int32, (D, 4 * D), 0)
    evens = (x0, x2, x4, x6)
    odds = (x1, x3, x5, x7)
    y_lo = None
    y_hi = None
    for k in range(4):
        ek = (lane == sub + k * D).astype(jnp.bfloat16)
        dlo = lax.dot_general(evens[k][...].astype(jnp.bfloat16), ek,
                              (((0,), (0,)), ((), ())),
                              preferred_element_type=jnp.float32)
        dhi = lax.dot_general(odds[k][...].astype(jnp.bfloat16), ek,
                              (((0,), (0,)), ((), ())),
                              preferred_element_type=jnp.float32)
        y_lo = dlo if y_lo is None else y_lo + dlo
        y_hi = dhi if y_hi is None else y_hi + dhi
    b_lo = _rne_bf16_bits(y_lo)
    b_hi = _rne_bf16_bits(y_hi)
    out_ref[...] = lax.bitcast_convert_type(b_lo | (b_hi << 16), jnp.int32)


def _make_pack(V: int, D: int):
    n_in_blocks = -(-V // _BQ)

    def in_spec(s):
        return pl.BlockSpec(
            (D, _BQ),
            lambda i, s=s: (0, jnp.minimum(s * _NBQ + i, n_in_blocks - 1)))

    return pl.pallas_call(
        _pack_kernel,
        grid=(_NBQ,),
        in_specs=[in_spec(s) for s in range(8)],
        out_specs=pl.BlockSpec((_BQ, 4 * D), lambda i: (i, 0)),
        out_shape=jax.ShapeDtypeStruct((_VQP2, 4 * D), jnp.int32),
        compiler_params=pltpu.CompilerParams(
            dimension_semantics=("arbitrary",)),
    )


def _make_gather(total_rows: int, Dq: int):
    rows_per_w = total_rows // _NW
    n_chunks = rows_per_w // _CHUNK

    @functools.partial(
        pl.kernel,
        mesh=plsc.VectorSubcoreMesh(core_axis_name="c", subcore_axis_name="s"),
        out_type=jax.ShapeDtypeStruct((total_rows, Dq), jnp.int32),
        scratch_types=[
            pltpu.VMEM((n_chunks, _CHUNK), jnp.int32),
            pltpu.VMEM((_NBUF, _CHUNK, Dq), jnp.int32),
            pltpu.SemaphoreType.DMA,
            pltpu.SemaphoreType.DMA,
        ],
    )
    def gather_k(idx_hbm, emb_hbm, out_hbm, idx_v, bufs, gsem, osem):
        wid = lax.axis_index("s") * _NC + lax.axis_index("c")
        pltpu.sync_copy(idx_hbm.at[wid], idx_v)
        base = wid * rows_per_w
        gcopies = [None] * n_chunks
        ocopies = [None] * n_chunks
        o_waited = [False] * n_chunks
        for j in range(n_chunks):
            if j >= _NBUF:
                ocopies[j - _NBUF].wait()
                o_waited[j - _NBUF] = True
            gcopies[j] = pltpu.async_copy(
                emb_hbm.at[idx_v.at[j]], bufs.at[j % _NBUF], gsem)
            jj = j - (_NBUF - 1)
            if jj >= 0:
                gcopies[jj].wait()
                ocopies[jj] = pltpu.async_copy(
                    bufs.at[jj % _NBUF],
                    out_hbm.at[pl.ds(base + jj * _CHUNK, _CHUNK)], osem)
        for jj in range(n_chunks - (_NBUF - 1), n_chunks):
            gcopies[jj].wait()
            ocopies[jj] = pltpu.async_copy(
                bufs.at[jj % _NBUF],
                out_hbm.at[pl.ds(base + jj * _CHUNK, _CHUNK)], osem)
        for jj in range(n_chunks):
            if not o_waited[jj]:
                ocopies[jj].wait()

    return gather_k


def _mlp_kernel(xn_ref, q_ref, m_ref, w1n_ref, w1e_ref, b1_ref, w2_ref,
                b2_ref, w3_ref, b3_ref, out_ref):
    bm = xn_ref.shape[1]
    h = lax.dot_general(xn_ref[...], w1n_ref[...], (((0,), (0,)), ((), ())),
                        preferred_element_type=jnp.float32)
    lane_grp = lax.broadcasted_iota(jnp.int32, (bm, 128), 1) >> 5
    for j in range(q_ref.shape[0]):
        w = q_ref[j]
        f_lo = lax.bitcast_convert_type(w << 16, jnp.float32)
        f_hi = lax.bitcast_convert_type(w & jnp.int32(-65536), jnp.float32)
        m = m_ref[j].reshape(bm, 1)
        p = jnp.where((m & 1) == 1, f_hi, f_lo)
        x = jnp.where(lane_grp == (m >> 1), p, 0.0)
        h += jnp.dot(x, w1e_ref[j], preferred_element_type=jnp.float32)
    h = jnp.maximum(h + b1_ref[...], 0.0)
    h = jnp.dot(h, w2_ref[...], preferred_element_type=jnp.float32)
    h = jnp.maximum(h + b2_ref[...], 0.0)
    o = jnp.dot(h, w3_ref[...], preferred_element_type=jnp.float32) + b3_ref[...]
    out_ref[...] = 1.0 / (1.0 + jnp.exp(-o))


def kernel(x_numeric, b1_idx, b2_idx, bowler_idx, emb, W1, bias1, W2, bias2,
           W3, bias3):
    B, IN = x_numeric.shape
    V, D = emb.shape
    H = W1.shape[1]
    total_rows = 3 * B
    Dq = 4 * D

    emb_q = _make_pack(V, D)(*([emb.T] * 8))

    idx_all = jnp.concatenate([b1_idx, b2_idx, bowler_idx])
    m_all = idx_all // _VQP2
    qidx = (idx_all - m_all * _VQP2).reshape(
        _NW, total_rows // (_NW * _CHUNK), _CHUNK)

    q = _make_gather(total_rows, Dq)(qidx, emb_q)
    q = q.reshape(3, B, Dq)

    w1n = W1[:IN]
    w1e = W1[IN:].reshape(3, D, H)
    w1e_exp = jnp.concatenate([w1e] * 4, axis=1)

    bm = 2048
    grid = (B // bm,)
    out = pl.pallas_call(
        _mlp_kernel,
        grid=grid,
        in_specs=[
            pl.BlockSpec((IN, bm), lambda i: (0, i)),
            pl.BlockSpec((3, bm, Dq), lambda i: (0, i, 0)),
            pl.BlockSpec((3, bm), lambda i: (0, i)),
            pl.BlockSpec((IN, H), lambda i: (0, 0)),
            pl.BlockSpec((3, Dq, H), lambda i: (0, 0, 0)),
            pl.BlockSpec((1, H), lambda i: (0, 0)),
            pl.BlockSpec((H, H), lambda i: (0, 0)),
            pl.BlockSpec((1, H), lambda i: (0, 0)),
            pl.BlockSpec((H, 1), lambda i: (0, 0)),
            pl.BlockSpec((1, 1), lambda i: (0, 0)),
        ],
        out_specs=pl.BlockSpec((bm, 1), lambda i: (i, 0)),
        out_shape=jax.ShapeDtypeStruct((B, 1), jnp.float32),
        compiler_params=pltpu.CompilerParams(
            dimension_semantics=("parallel",)),
    )(x_numeric.T, q, m_all.reshape(3, B), w1n, w1e_exp,
      bias1.reshape(1, H), W2, bias2.reshape(1, H), W3, bias3.reshape(1, 1))
    return out.reshape(B)


# speedup vs baseline: 1.0092x; 1.0092x over previous
import functools

import jax
import jax.numpy as jnp
from jax import lax
from jax.experimental import pallas as pl
from jax.experimental.pallas import tpu as pltpu
from jax.experimental.pallas import tpu_sc as plsc

_NC = 2
_NS = 16
_NW = _NC * _NS
_CHUNK = 128
_NBUF = 4

_BQ = 8192
_NBQ = 16
_VQP2 = _BQ * _NBQ


def _rne_bf16_bits(y):
    u = lax.bitcast_convert_type(y, jnp.uint32)
    return (u + 0x7FFF + ((u >> 16) & 1)) >> 16


def _pack_kernel(x0, x1, x2, x3, x4, x5, x6, x7, out_ref):
    D = x0.shape[0]
    lane = lax.broadcasted_iota(jnp.int32, (D, 4 * D), 1)
    sub = lax.broadcasted_iota(jnp.int32, (D, 4 * D), 0)
    evens = (x0, x2, x4, x6)
    odds = (x1, x3, x5, x7)
    y_lo = None
    y_hi = None
    for k in range(4):
        ek = (lane == sub + k * D).astype(jnp.bfloat16)
        dlo = lax.dot_general(evens[k][...].astype(jnp.bfloat16), ek,
                              (((0,), (0,)), ((), ())),
                              preferred_element_type=jnp.float32)
        dhi = lax.dot_general(odds[k][...].astype(jnp.bfloat16), ek,
                              (((0,), (0,)), ((), ())),
                              preferred_element_type=jnp.float32)
        y_lo = dlo if y_lo is None else y_lo + dlo
        y_hi = dhi if y_hi is None else y_hi + dhi
    b_lo = _rne_bf16_bits(y_lo)
    b_hi = _rne_bf16_bits(y_hi)
    out_ref[...] = lax.bitcast_convert_type(b_lo | (b_hi << 16), jnp.int32)


def _make_pack(V: int, D: int):
    n_in_blocks = -(-V // _BQ)

    def in_spec(s):
        return pl.BlockSpec(
            (D, _BQ),
            lambda i, s=s: (0, jnp.minimum(s * _NBQ + i, n_in_blocks - 1)))

    return pl.pallas_call(
        _pack_kernel,
        grid=(_NBQ,),
        in_specs=[in_spec(s) for s in range(8)],
        out_specs=pl.BlockSpec((_BQ, 4 * D), lambda i: (i, 0)),
        out_shape=jax.ShapeDtypeStruct((_VQP2, 4 * D), jnp.int32),
        compiler_params=pltpu.CompilerParams(
            dimension_semantics=("arbitrary",)),
    )


def _make_gather(total_rows: int, Dq: int):
    rows_per_w = total_rows // _NW
    n_chunks = rows_per_w // _CHUNK

    @functools.partial(
        pl.kernel,
        mesh=plsc.VectorSubcoreMesh(core_axis_name="c", subcore_axis_name="s"),
        out_type=jax.ShapeDtypeStruct((total_rows, Dq), jnp.int32),
        scratch_types=[
            pltpu.VMEM((n_chunks, _CHUNK), jnp.int32),
            pltpu.VMEM((_NBUF, _CHUNK, Dq), jnp.int32),
            pltpu.SemaphoreType.DMA,
            pltpu.SemaphoreType.DMA,
        ],
    )
    def gather_k(idx_hbm, emb_hbm, out_hbm, idx_v, bufs, gsem, osem):
        wid = lax.axis_index("s") * _NC + lax.axis_index("c")
        pltpu.sync_copy(idx_hbm.at[wid], idx_v)
        base = wid * rows_per_w
        gcopies = [None] * n_chunks
        ocopies = [None] * n_chunks
        o_waited = [False] * n_chunks
        for j in range(n_chunks):
            if j >= _NBUF:
                ocopies[j - _NBUF].wait()
                o_waited[j - _NBUF] = True
            gcopies[j] = pltpu.async_copy(
                emb_hbm.at[idx_v.at[j]], bufs.at[j % _NBUF], gsem)
            jj = j - (_NBUF - 1)
            if jj >= 0:
                gcopies[jj].wait()
                ocopies[jj] = pltpu.async_copy(
                    bufs.at[jj % _NBUF],
                    out_hbm.at[pl.ds(base + jj * _CHUNK, _CHUNK)], osem)
        for jj in range(n_chunks - (_NBUF - 1), n_chunks):
            gcopies[jj].wait()
            ocopies[jj] = pltpu.async_copy(
                bufs.at[jj % _NBUF],
                out_hbm.at[pl.ds(base + jj * _CHUNK, _CHUNK)], osem)
        for jj in range(n_chunks):
            if not o_waited[jj]:
                ocopies[jj].wait()

    return gather_k


def _mlp_kernel(xn_ref, q_ref, m_ref, w1n_ref, w1e_ref, b1_ref, w2_ref,
                b2_ref, w3_ref, b3_ref, out_ref):
    bm = xn_ref.shape[1]
    h = lax.dot_general(xn_ref[...], w1n_ref[...], (((0,), (0,)), ((), ())),
                        preferred_element_type=jnp.float32)
    lane_grp = lax.broadcasted_iota(jnp.int32, (bm, 128), 1) >> 5
    for j in range(q_ref.shape[0]):
        w = q_ref[j]
        f_lo = lax.bitcast_convert_type(w << 16, jnp.float32)
        f_hi = lax.bitcast_convert_type(w & jnp.int32(-65536), jnp.float32)
        m = m_ref[j].reshape(bm, 1)
        p = jnp.where((m & 1) == 1, f_hi, f_lo)
        x = jnp.where(lane_grp == (m >> 1), p, 0.0)
        h += jnp.dot(x, w1e_ref[j], preferred_element_type=jnp.float32)
    h = jnp.maximum(h + b1_ref[...], 0.0)
    h = jnp.dot(h, w2_ref[...], preferred_element_type=jnp.float32)
    h = jnp.maximum(h + b2_ref[...], 0.0)
    o = jnp.dot(h, w3_ref[...], preferred_element_type=jnp.float32) + b3_ref[...]
    out_ref[...] = 1.0 / (1.0 + jnp.exp(-o))


def kernel(x_numeric, b1_idx, b2_idx, bowler_idx, emb, W1, bias1, W2, bias2,
           W3, bias3):
    B, IN = x_numeric.shape
    V, D = emb.shape
    H = W1.shape[1]
    total_rows = 3 * B
    Dq = 4 * D

    emb_q = _make_pack(V, D)(*([emb.T] * 8))

    idx_all = jnp.concatenate([b1_idx, b2_idx, bowler_idx])
    m_all = idx_all // _VQP2
    qidx = (idx_all - m_all * _VQP2).reshape(
        _NW, total_rows // (_NW * _CHUNK), _CHUNK)

    q = _make_gather(total_rows, Dq)(qidx, emb_q)
    q = q.reshape(3, B, Dq)

    w1n = W1[:IN]
    w1e = W1[IN:].reshape(3, D, H)
    w1e_exp = jnp.concatenate([w1e] * 4, axis=1)

    bm = 4096
    grid = (B // bm,)
    out = pl.pallas_call(
        _mlp_kernel,
        grid=grid,
        in_specs=[
            pl.BlockSpec((IN, bm), lambda i: (0, i)),
            pl.BlockSpec((3, bm, Dq), lambda i: (0, i, 0)),
            pl.BlockSpec((3, bm), lambda i: (0, i)),
            pl.BlockSpec((IN, H), lambda i: (0, 0)),
            pl.BlockSpec((3, Dq, H), lambda i: (0, 0, 0)),
            pl.BlockSpec((1, H), lambda i: (0, 0)),
            pl.BlockSpec((H, H), lambda i: (0, 0)),
            pl.BlockSpec((1, H), lambda i: (0, 0)),
            pl.BlockSpec((H, 1), lambda i: (0, 0)),
            pl.BlockSpec((1, 1), lambda i: (0, 0)),
        ],
        out_specs=pl.BlockSpec((bm, 1), lambda i: (i, 0)),
        out_shape=jax.ShapeDtypeStruct((B, 1), jnp.float32),
        compiler_params=pltpu.CompilerParams(
            dimension_semantics=("parallel",)),
    )(x_numeric.T, q, m_all.reshape(3, B), w1n, w1e_exp,
      bias1.reshape(1, H), W2, bias2.reshape(1, H), W3, bias3.reshape(1, 1))
    return out.reshape(B)
